# Initial kernel scaffold; baseline (speedup 1.0000x reference)
import jax
import jax.numpy as jnp
from jax.experimental import pallas as pl


def kernel(coords1, feats1, coords2, feats2, W1, b1, W2, b2, W3, b3, W4, b4, W5, b5):
    raise NotImplementedError("write your pallas kernel here")


# speedup vs baseline: 8.7570x; 8.7570x over previous
import functools

import jax
import jax.numpy as jnp
from jax import lax
from jax.experimental import pallas as pl
from jax.experimental.pallas import tpu as pltpu
from jax.experimental.pallas import tpu_sc as plsc

KNN_K = 8
N1 = 8192
N2 = 8192
CF = 64
BQ = 256
GRID = N2 // BQ
TW = 128
IDX_MASK = 8191
INT_MAX = 2**31 - 1

SC_NC = 2
SC_NS = 16
SC_NW = SC_NC * SC_NS
SC_ROWS = KNN_K * N2
SC_PER_W = SC_ROWS // SC_NW
SC_CH = 128
SC_NCH = SC_PER_W // SC_CH


def _knn_body(c2p_ref, c1t_ref, idx_ref):
    q = c2p_ref[...]
    c1t = c1t_ref[...]
    qr = jnp.dot(q, c1t, preferred_element_type=jnp.float32)
    q2 = jnp.sum(q * q, axis=1, keepdims=True)
    r2 = jnp.sum(c1t * c1t, axis=0, keepdims=True)
    d2 = jnp.maximum(q2 + r2 - 2.0 * qr, 0.0)
    key = lax.bitcast_convert_type(d2, jnp.int32)
    col = lax.broadcasted_iota(jnp.int32, (BQ, N1), 1)
    packed = jnp.bitwise_or(jnp.bitwise_and(key, ~IDX_MASK), col)
    idxs = []
    for _ in range(KNN_K):
        m = jnp.min(packed, axis=1, keepdims=True)
        idxs.append(jnp.bitwise_and(m, IDX_MASK))
        packed = jnp.where(packed == m, jnp.int32(INT_MAX), packed)
    idx_ref[...] = jnp.concatenate(idxs, axis=1)


def _knn_topk(coords2p, coords1t):
    return pl.pallas_call(
        _knn_body,
        grid=(GRID,),
        in_specs=[
            pl.BlockSpec((BQ, 8), lambda i: (i, 0)),
            pl.BlockSpec((8, N1), lambda i: (0, 0)),
        ],
        out_specs=pl.BlockSpec((BQ, KNN_K), lambda i: (i, 0)),
        out_shape=jax.ShapeDtypeStruct((N2, KNN_K), jnp.int32),
    )(coords2p, coords1t)


def _sc_gather_body(table_hbm, idx_hbm, out_hbm, idx_v, buf0, buf1, sem0, sem1):
    wid = lax.axis_index("s") * SC_NC + lax.axis_index("c")
    base = wid * SC_PER_W
    pltpu.sync_copy(idx_hbm.at[pl.ds(base, SC_PER_W)], idx_v)
    bufs = (buf0, buf1)
    sems = (sem0, sem1)

    def start(c):
        return pltpu.async_copy(
            table_hbm.at[idx_v.at[pl.ds(c * SC_CH, SC_CH)]],
            bufs[c % 2], sems[c % 2])

    cps = [start(0), None]
    for c in range(SC_NCH):
        if c + 1 < SC_NCH:
            cps[(c + 1) % 2] = start(c + 1)
        cps[c % 2].wait()
        pltpu.sync_copy(bufs[c % 2], out_hbm.at[pl.ds(base + c * SC_CH, SC_CH)])


def _sc_gather(table, idx_flat):
    mesh = plsc.VectorSubcoreMesh(core_axis_name="c", subcore_axis_name="s")
    k = functools.partial(
        pl.kernel,
        mesh=mesh,
        out_type=jax.ShapeDtypeStruct((SC_ROWS, TW), jnp.float32),
        scratch_types=[
            pltpu.VMEM((SC_PER_W,), jnp.int32),
            pltpu.VMEM((SC_CH, TW), jnp.float32),
            pltpu.VMEM((SC_CH, TW), jnp.float32),
            pltpu.SemaphoreType.DMA,
            pltpu.SemaphoreType.DMA,
        ],
    )(_sc_gather_body)
    return k(table, idx_flat)


def _mlp1_body(g_ref, qp_ref, w1x_ref, w1f_ref, w1c_ref, b1_ref,
               w2_ref, b2_ref, w3t_ref, b3_ref,
               logit_ref, stats_ref, m_s, s_s):
    pid = pl.program_id(0)
    qp = qp_ref[...]

    gxyz = []
    invd = []
    normsum = jnp.zeros((BQ, 1), jnp.float32)
    for k in range(KNN_K):
        gk = g_ref[k]
        gx = gk[:, CF:CF + 8] - qp
        dist = jnp.sqrt(jnp.sum(gx * gx, axis=1, keepdims=True))
        dist = jnp.maximum(dist, 1e-10)
        iv = 1.0 / dist
        gxyz.append(gx)
        invd.append(iv)
        normsum = normsum + iv
    rnorm = 1.0 / normsum

    logits = []
    for k in range(KNN_K):
        gk = g_ref[k]
        nf = gk[:, :CF]
        cw = invd[k] * rnorm
        h1 = (jnp.dot(gxyz[k], w1x_ref[...], preferred_element_type=jnp.float32)
              + jnp.dot(nf, w1f_ref[...], preferred_element_type=jnp.float32)
              + cw * w1c_ref[...] + b1_ref[...])
        h1 = jnp.maximum(h1, 0.0)
        h2 = jnp.dot(h1, w2_ref[...], preferred_element_type=jnp.float32) + b2_ref[...]
        h2 = jnp.maximum(h2, 0.0)
        lg = jnp.sum(h2 * w3t_ref[...], axis=1, keepdims=True) + b3_ref[...]
        logits.append(lg)
    lblk = jnp.concatenate(logits, axis=1)
    logit_ref[...] = lblk

    @pl.when(pid == 0)
    def _():
        m_s[...] = jnp.full((1, KNN_K), -jnp.inf, jnp.float32)
        s_s[...] = jnp.zeros((1, KNN_K), jnp.float32)

    m_blk = jnp.max(lblk, axis=0, keepdims=True)
    m_old = m_s[...]
    m_new = jnp.maximum(m_old, m_blk)
    s_s[...] = (s_s[...] * jnp.exp(m_old - m_new)
                + jnp.sum(jnp.exp(lblk - m_new), axis=0, keepdims=True))
    m_s[...] = m_new

    @pl.when(pid == GRID - 1)
    def _():
        stats_ref[...] = jnp.concatenate([m_s[...], s_s[...]], axis=0)


def _mlp1_logits(g, coords2p, w1x, w1f, w1c, b1, w2, b2, w3t, b3):
    return pl.pallas_call(
        _mlp1_body,
        grid=(GRID,),
        in_specs=[
            pl.BlockSpec((KNN_K, BQ, TW), lambda i: (0, i, 0)),
            pl.BlockSpec((BQ, 8), lambda i: (i, 0)),
            pl.BlockSpec((8, 128), lambda i: (0, 0)),
            pl.BlockSpec((CF, 128), lambda i: (0, 0)),
            pl.BlockSpec((1, 128), lambda i: (0, 0)),
            pl.BlockSpec((1, 128), lambda i: (0, 0)),
            pl.BlockSpec((128, CF), lambda i: (0, 0)),
            pl.BlockSpec((1, CF), lambda i: (0, 0)),
            pl.BlockSpec((1, CF), lambda i: (0, 0)),
            pl.BlockSpec((1, 1), lambda i: (0, 0)),
        ],
        out_specs=[
            pl.BlockSpec((BQ, KNN_K), lambda i: (i, 0)),
            pl.BlockSpec((2, KNN_K), lambda i: (0, 0)),
        ],
        out_shape=[
            jax.ShapeDtypeStruct((N2, KNN_K), jnp.float32),
            jax.ShapeDtypeStruct((2, KNN_K), jnp.float32),
        ],
        scratch_shapes=[
            pltpu.VMEM((1, KNN_K), jnp.float32),
            pltpu.VMEM((1, KNN_K), jnp.float32),
        ],
    )(g, coords2p, w1x, w1f, w1c, b1, w2, b2, w3t, b3)


def _agg_body(stats_ref, l_ref, g_ref, f2_ref, w4_ref, b4_ref, w5_ref, b5_ref,
              pred_ref, resid_ref):
    m = stats_ref[0:1, :]
    s = stats_ref[1:2, :]
    w = jnp.exp(l_ref[...] - m) / s
    interp = jnp.zeros((BQ, CF), jnp.float32)
    for k in range(KNN_K):
        interp = interp + w[:, k:k + 1] * g_ref[k][:, :CF]
    h = jnp.dot(interp, w4_ref[...], preferred_element_type=jnp.float32) + b4_ref[...]
    h = jnp.maximum(h, 0.0)
    pred = jnp.dot(h, w5_ref[...], preferred_element_type=jnp.float32) + b5_ref[...]
    pred_ref[...] = pred
    resid_ref[...] = f2_ref[...] - pred


def _aggregate(stats, logits, g, feats2, w4, b4, w5, b5):
    return pl.pallas_call(
        _agg_body,
        grid=(GRID,),
        in_specs=[
            pl.BlockSpec((2, KNN_K), lambda i: (0, 0)),
            pl.BlockSpec((BQ, KNN_K), lambda i: (i, 0)),
            pl.BlockSpec((KNN_K, BQ, TW), lambda i: (0, i, 0)),
            pl.BlockSpec((BQ, CF), lambda i: (i, 0)),
            pl.BlockSpec((CF, 256), lambda i: (0, 0)),
            pl.BlockSpec((1, 256), lambda i: (0, 0)),
            pl.BlockSpec((256, CF), lambda i: (0, 0)),
            pl.BlockSpec((1, CF), lambda i: (0, 0)),
        ],
        out_specs=[
            pl.BlockSpec((BQ, CF), lambda i: (i, 0)),
            pl.BlockSpec((BQ, CF), lambda i: (i, 0)),
        ],
        out_shape=[
            jax.ShapeDtypeStruct((N2, CF), jnp.float32),
            jax.ShapeDtypeStruct((N2, CF), jnp.float32),
        ],
    )(stats, logits, g, feats2, w4, b4, w5, b5)


def kernel(coords1, feats1, coords2, feats2,
           W1, b1, W2, b2, W3, b3, W4, b4, W5, b5):
    coords2p = jnp.pad(coords2, ((0, 0), (0, 5)))
    coords1t = jnp.pad(coords1, ((0, 0), (0, 5))).T

    knn = _knn_topk(coords2p, coords1t)

    table = jnp.concatenate(
        [feats1, coords1, jnp.zeros((N1, TW - CF - 3), jnp.float32)], axis=1)
    idx_flat = knn.T.reshape(-1)
    g = _sc_gather(table, idx_flat).reshape(KNN_K, N2, TW)

    w1x = jnp.pad(W1[0:3], ((0, 5), (0, 0)))
    w1f = W1[3:67]
    w1c = W1[67:68]
    logits, stats = _mlp1_logits(
        g, coords2p, w1x, w1f, w1c, b1.reshape(1, -1),
        W2, b2.reshape(1, -1), W3.reshape(1, -1), b3.reshape(1, 1))

    pred, resid = _aggregate(
        stats, logits, g, feats2, W4, b4.reshape(1, -1), W5, b5.reshape(1, -1))
    return pred, resid


# speedup vs baseline: 11.7806x; 1.3453x over previous
import functools

import jax
import jax.numpy as jnp
from jax import lax
from jax.experimental import pallas as pl
from jax.experimental.pallas import tpu as pltpu
from jax.experimental.pallas import tpu_sc as plsc

KNN_K = 8
N1 = 8192
N2 = 8192
CF = 64
BQ = 256
GRID = N2 // BQ
TW = 128
IDX_MASK = 8191
INT_MAX = 2**31 - 1

SC_NC = 2
SC_NS = 16
SC_NW = SC_NC * SC_NS
SC_ROWS = KNN_K * N2
SC_PER_W = SC_ROWS // SC_NW
SC_CH = 128
SC_NCH = SC_PER_W // SC_CH


def _knn_body(qs_ref, c1t_ref, idx_ref):
    qs = qs_ref[...]
    c1t = c1t_ref[...]
    mm = jnp.dot(qs, c1t, preferred_element_type=jnp.float32)
    r2 = jnp.sum(c1t * c1t, axis=0, keepdims=True)
    d2s = mm + (r2 + 65536.0)
    key = lax.bitcast_convert_type(d2s, jnp.int32)
    col = lax.broadcasted_iota(jnp.int32, (BQ, N1), 1)
    packed = lax.bitcast_convert_type(
        jnp.bitwise_or(jnp.bitwise_and(key, ~IDX_MASK), col), jnp.float32)
    idxs = []
    m = jnp.min(packed, axis=1, keepdims=True)
    for k in range(KNN_K):
        idxs.append(jnp.bitwise_and(
            lax.bitcast_convert_type(m, jnp.int32), IDX_MASK))
        if k + 1 < KNN_K:
            packed = jnp.where(packed == m, jnp.inf, packed)
            m = jnp.min(packed, axis=1, keepdims=True)
    idx_ref[...] = jnp.concatenate(idxs, axis=1)


def _knn_topk(coords2s, coords1t):
    return pl.pallas_call(
        _knn_body,
        grid=(GRID,),
        in_specs=[
            pl.BlockSpec((BQ, 8), lambda i: (i, 0)),
            pl.BlockSpec((8, N1), lambda i: (0, 0)),
        ],
        out_specs=pl.BlockSpec((BQ, KNN_K), lambda i: (i, 0)),
        out_shape=jax.ShapeDtypeStruct((N2, KNN_K), jnp.int32),
    )(coords2s, coords1t)


def _sc_gather_body(table_hbm, idx_hbm, out_hbm, idx_v, buf0, buf1, sem0, sem1):
    wid = lax.axis_index("s") * SC_NC + lax.axis_index("c")
    base = wid * SC_PER_W
    pltpu.sync_copy(idx_hbm.at[pl.ds(base, SC_PER_W)], idx_v)
    bufs = (buf0, buf1)
    sems = (sem0, sem1)

    def start(c):
        return pltpu.async_copy(
            table_hbm.at[idx_v.at[pl.ds(c * SC_CH, SC_CH)]],
            bufs[c % 2], sems[c % 2])

    cps = [start(0), None]
    for c in range(SC_NCH):
        if c + 1 < SC_NCH:
            cps[(c + 1) % 2] = start(c + 1)
        cps[c % 2].wait()
        pltpu.sync_copy(bufs[c % 2], out_hbm.at[pl.ds(base + c * SC_CH, SC_CH)])


def _sc_gather(table, idx_flat):
    mesh = plsc.VectorSubcoreMesh(core_axis_name="c", subcore_axis_name="s")
    k = functools.partial(
        pl.kernel,
        mesh=mesh,
        out_type=jax.ShapeDtypeStruct((SC_ROWS, TW), jnp.float32),
        scratch_types=[
            pltpu.VMEM((SC_PER_W,), jnp.int32),
            pltpu.VMEM((SC_CH, TW), jnp.float32),
            pltpu.VMEM((SC_CH, TW), jnp.float32),
            pltpu.SemaphoreType.DMA,
            pltpu.SemaphoreType.DMA,
        ],
    )(_sc_gather_body)
    return k(table, idx_flat)


def _mlp_body(g_ref, qp_ref, f2_ref, w1x_ref, w1f_ref, w1c_ref, b1_ref,
              w2_ref, b2_ref, w3t_ref, b3_ref, w4_ref, b4_ref, w5_ref, b5_ref,
              pred_ref, resid_ref, l_scr, m_s, s_s):
    p = pl.program_id(0)
    i = pl.program_id(1)
    row0 = pl.multiple_of(i * BQ, BQ)

    @pl.when(p == 0)
    def _phase_logits():
        qp = qp_ref[...]
        gxyz = []
        invd = []
        normsum = jnp.zeros((BQ, 1), jnp.float32)
        for k in range(KNN_K):
            gk = g_ref[k]
            gx = gk[:, CF:CF + 8] - qp
            dist = jnp.sqrt(jnp.sum(gx * gx, axis=1, keepdims=True))
            dist = jnp.maximum(dist, 1e-10)
            iv = 1.0 / dist
            gxyz.append(gx)
            invd.append(iv)
            normsum = normsum + iv
        rnorm = 1.0 / normsum

        logits = []
        for k in range(KNN_K):
            gk = g_ref[k]
            nf = gk[:, :CF]
            cw = invd[k] * rnorm
            h1 = (jnp.dot(gxyz[k], w1x_ref[...], preferred_element_type=jnp.float32)
                  + jnp.dot(nf, w1f_ref[...], preferred_element_type=jnp.float32)
                  + cw * w1c_ref[...] + b1_ref[...])
            h1 = jnp.maximum(h1, 0.0)
            h2 = jnp.dot(h1, w2_ref[...], preferred_element_type=jnp.float32) + b2_ref[...]
            h2 = jnp.maximum(h2, 0.0)
            lg = jnp.sum(h2 * w3t_ref[...], axis=1, keepdims=True) + b3_ref[...]
            logits.append(lg)
        lblk = jnp.concatenate(logits, axis=1)
        l_scr[pl.ds(row0, BQ), :] = lblk

        @pl.when(i == 0)
        def _():
            m_s[...] = jnp.full((1, KNN_K), -jnp.inf, jnp.float32)
            s_s[...] = jnp.zeros((1, KNN_K), jnp.float32)

        m_blk = jnp.max(lblk, axis=0, keepdims=True)
        m_old = m_s[...]
        m_new = jnp.maximum(m_old, m_blk)
        s_s[...] = (s_s[...] * jnp.exp(m_old - m_new)
                    + jnp.sum(jnp.exp(lblk - m_new), axis=0, keepdims=True))
        m_s[...] = m_new

    @pl.when(p == 1)
    def _phase_aggregate():
        w = jnp.exp(l_scr[pl.ds(row0, BQ), :] - m_s[...]) / s_s[...]
        interp = jnp.zeros((BQ, CF), jnp.float32)
        for k in range(KNN_K):
            interp = interp + w[:, k:k + 1] * g_ref[k][:, :CF]
        h = jnp.dot(interp, w4_ref[...], preferred_element_type=jnp.float32) + b4_ref[...]
        h = jnp.maximum(h, 0.0)
        pred = jnp.dot(h, w5_ref[...], preferred_element_type=jnp.float32) + b5_ref[...]
        pred_ref[...] = pred
        resid_ref[...] = f2_ref[...] - pred


def _mlp_pipeline(g, coords2p, feats2, w1x, w1f, w1c, b1, w2, b2, w3t, b3,
                  w4, b4, w5, b5):
    const = lambda p, i: (0, 0)
    return pl.pallas_call(
        _mlp_body,
        grid=(2, GRID),
        in_specs=[
            pl.BlockSpec((KNN_K, BQ, TW), lambda p, i: (0, i, 0)),
            pl.BlockSpec((BQ, 8), lambda p, i: (i, 0)),
            pl.BlockSpec((BQ, CF), lambda p, i: (i, 0)),
            pl.BlockSpec((8, 128), const),
            pl.BlockSpec((CF, 128), const),
            pl.BlockSpec((1, 128), const),
            pl.BlockSpec((1, 128), const),
            pl.BlockSpec((128, CF), const),
            pl.BlockSpec((1, CF), const),
            pl.BlockSpec((1, CF), const),
            pl.BlockSpec((1, 1), const),
            pl.BlockSpec((CF, 256), const),
            pl.BlockSpec((1, 256), const),
            pl.BlockSpec((256, CF), const),
            pl.BlockSpec((1, CF), const),
        ],
        out_specs=[
            pl.BlockSpec((BQ, CF), lambda p, i: (i, 0)),
            pl.BlockSpec((BQ, CF), lambda p, i: (i, 0)),
        ],
        out_shape=[
            jax.ShapeDtypeStruct((N2, CF), jnp.float32),
            jax.ShapeDtypeStruct((N2, CF), jnp.float32),
        ],
        scratch_shapes=[
            pltpu.VMEM((N2, KNN_K), jnp.float32),
            pltpu.VMEM((1, KNN_K), jnp.float32),
            pltpu.VMEM((1, KNN_K), jnp.float32),
        ],
    )(g, coords2p, feats2, w1x, w1f, w1c, b1, w2, b2, w3t, b3, w4, b4, w5, b5)


def kernel(coords1, feats1, coords2, feats2,
           W1, b1, W2, b2, W3, b3, W4, b4, W5, b5):
    coords2p = jnp.pad(coords2, ((0, 0), (0, 5)))
    coords1t = jnp.pad(coords1, ((0, 0), (0, 5))).T

    knn = _knn_topk(-2.0 * coords2p, coords1t)

    table = jnp.concatenate(
        [feats1, coords1, jnp.zeros((N1, TW - CF - 3), jnp.float32)], axis=1)
    idx_flat = knn.T.reshape(-1)
    g = _sc_gather(table, idx_flat).reshape(KNN_K, N2, TW)

    w1x = jnp.pad(W1[0:3], ((0, 5), (0, 0)))
    w1f = W1[3:67]
    w1c = W1[67:68]
    pred, resid = _mlp_pipeline(
        g, coords2p, feats2, w1x, w1f, w1c, b1.reshape(1, -1),
        W2, b2.reshape(1, -1), W3.reshape(1, -1), b3.reshape(1, 1),
        W4, b4.reshape(1, -1), W5, b5.reshape(1, -1))
    return pred, resid
